# baseline (speedup 1.0000x reference)
import functools

import jax
import jax.numpy as jnp
from jax import lax
from jax.experimental import pallas as pl
from jax.experimental.pallas import tpu as pltpu, tpu_sc as plsc

N = 10000
D_IN = 128
D_HID = 128
D_OUT = 40
D_OUT_PAD = 128

NC = 2
NS = 16
NW = NC * NS
CH = 128

N_PAD = 10240
ROWS_PER_TILE = N_PAD // NS

_MESH = plsc.VectorSubcoreMesh(
    core_axis_name="c", subcore_axis_name="s", num_cores=NC, num_subcores=NS
)


def _unpack_chunk(pk_v, j, s_buf, d_buf):
    for k in range(CH // 16):
        v = pk_v[j, pl.ds(k * 16, 16)]
        s_buf[pl.ds(k * 16, 16)] = v >> 14
        d_buf[pl.ds(k * 16, 16)] = v & 0x3FFF


def _seg_body(t_chunks, d, hs_hbm, pk_hbm, z_hbm, out_hbm,
              pk_v, sa, sb, da, db, ra, rb, sem_a, sem_b, acc):
    c = lax.axis_index("c")
    s = lax.axis_index("s")
    w = c * NS + s
    r0 = s * ROWS_PER_TILE
    pltpu.sync_copy(z_hbm.at[pl.ds(r0, ROWS_PER_TILE)],
                    acc.at[pl.ds(r0, ROWS_PER_TILE)])
    pltpu.sync_copy(pk_hbm.at[w], pk_v)
    plsc.subcore_barrier()

    def body(j, carry):
        _unpack_chunk(pk_v, j, sa, da)
        pltpu.async_copy(hs_hbm.at[sa], ra, sem_a).wait()
        pltpu.sync_copy(ra, acc.at[da], add=True)
        return carry

    lax.fori_loop(0, t_chunks, body, 0, unroll=False)
    plsc.subcore_barrier()
    pltpu.sync_copy(acc.at[pl.ds(r0, ROWS_PER_TILE)],
                    out_hbm.at[c, pl.ds(r0, ROWS_PER_TILE)])


def _make_seg_kernel(t_chunks, d):
    return pl.kernel(
        functools.partial(_seg_body, t_chunks, d),
        out_type=jax.ShapeDtypeStruct((NC, N_PAD, d), jnp.float32),
        mesh=_MESH,
        scratch_types=[
            pltpu.VMEM((t_chunks, CH), jnp.int32),
            pltpu.VMEM((CH,), jnp.int32),
            pltpu.VMEM((CH,), jnp.int32),
            pltpu.VMEM((CH,), jnp.int32),
            pltpu.VMEM((CH,), jnp.int32),
            pltpu.VMEM((CH, d), jnp.float32),
            pltpu.VMEM((CH, d), jnp.float32),
            pltpu.SemaphoreType.DMA,
            pltpu.SemaphoreType.DMA,
            pltpu.VMEM_SHARED((N_PAD, d), jnp.float32),
        ],
    )


def _deg_body(t_chunks, pk_hbm, ones_hbm, z_hbm, out_hbm,
              pk_v, sa, da, ones_v, acc):
    c = lax.axis_index("c")
    s = lax.axis_index("s")
    w = c * NS + s
    r0 = s * ROWS_PER_TILE
    pltpu.sync_copy(z_hbm.at[pl.ds(r0, ROWS_PER_TILE)],
                    acc.at[pl.ds(r0, ROWS_PER_TILE)])
    pltpu.sync_copy(ones_hbm, ones_v)
    pltpu.sync_copy(pk_hbm.at[w], pk_v)
    plsc.subcore_barrier()

    def body(j, carry):
        _unpack_chunk(pk_v, j, sa, da)
        pltpu.sync_copy(ones_v, acc.at[da], add=True)
        return carry

    lax.fori_loop(0, t_chunks, body, 0, unroll=False)
    plsc.subcore_barrier()
    pltpu.sync_copy(acc.at[pl.ds(r0, ROWS_PER_TILE)],
                    out_hbm.at[c, pl.ds(r0, ROWS_PER_TILE)])


def _make_deg_kernel(t_chunks):
    return pl.kernel(
        functools.partial(_deg_body, t_chunks),
        out_type=jax.ShapeDtypeStruct((NC, N_PAD, 128), jnp.float32),
        mesh=_MESH,
        scratch_types=[
            pltpu.VMEM((t_chunks, CH), jnp.int32),
            pltpu.VMEM((CH,), jnp.int32),
            pltpu.VMEM((CH,), jnp.int32),
            pltpu.VMEM((CH, 128), jnp.float32),
            pltpu.VMEM_SHARED((N_PAD, 128), jnp.float32),
        ],
    )


_BT = 1024


def _tc1_body(x_ref, w_ref, dp0_ref, dp1_ref, hs_ref, dinv_ref):
    i = pl.program_id(0)
    deg = dp0_ref[...] + dp1_ref[...] + 1.0
    dinv = lax.rsqrt(deg)
    rows = i * _BT + lax.broadcasted_iota(jnp.int32, (_BT,), 0)
    mask = (rows < N).astype(jnp.float32)
    h = jnp.dot(x_ref[...], w_ref[...], preferred_element_type=jnp.float32)
    hs_ref[...] = h * (dinv * mask)[:, None]
    dinv_ref[...] = dinv


def _tc1(x_pad, w1, dp0, dp1):
    grid = (N_PAD // _BT,)
    return pl.pallas_call(
        _tc1_body,
        grid=grid,
        in_specs=[
            pl.BlockSpec((_BT, D_IN), lambda i: (i, 0)),
            pl.BlockSpec((D_IN, D_HID), lambda i: (0, 0)),
            pl.BlockSpec((_BT,), lambda i: (i,)),
            pl.BlockSpec((_BT,), lambda i: (i,)),
        ],
        out_specs=[
            pl.BlockSpec((_BT, D_HID), lambda i: (i, 0)),
            pl.BlockSpec((_BT,), lambda i: (i,)),
        ],
        out_shape=[
            jax.ShapeDtypeStruct((N_PAD, D_HID), jnp.float32),
            jax.ShapeDtypeStruct((N_PAD,), jnp.float32),
        ],
    )(x_pad, w1, dp0, dp1)


def _tc_layer_body(s_ref, hs_ref, dinv_ref, b_ref, w_ref, out_ref):
    i = pl.program_id(0)
    dinv = dinv_ref[...]
    u = dinv[:, None] * (s_ref[0] + s_ref[1] + hs_ref[...]) + b_ref[...][None, :]
    x2 = jnp.maximum(u, 0.0)
    rows = i * _BT + lax.broadcasted_iota(jnp.int32, (_BT,), 0)
    mask = (rows < N).astype(jnp.float32)
    h = jnp.dot(x2, w_ref[...], preferred_element_type=jnp.float32)
    out_ref[...] = h * (dinv * mask)[:, None]


def _tc_layer(s_part, hs_prev, dinv, b, w):
    d_in = hs_prev.shape[1]
    d_out = w.shape[1]
    grid = (N_PAD // _BT,)
    return pl.pallas_call(
        _tc_layer_body,
        grid=grid,
        in_specs=[
            pl.BlockSpec((NC, _BT, d_in), lambda i: (0, i, 0)),
            pl.BlockSpec((_BT, d_in), lambda i: (i, 0)),
            pl.BlockSpec((_BT,), lambda i: (i,)),
            pl.BlockSpec((d_in,), lambda i: (0,)),
            pl.BlockSpec((d_in, d_out), lambda i: (0, 0)),
        ],
        out_specs=pl.BlockSpec((_BT, d_out), lambda i: (i, 0)),
        out_shape=jax.ShapeDtypeStruct((N_PAD, d_out), jnp.float32),
    )(s_part, hs_prev, dinv, b, w)


def _tc_final_body(s_ref, hs_ref, dinv_ref, b_ref, out_ref):
    dinv = dinv_ref[...]
    u = dinv[:, None] * (s_ref[0] + s_ref[1] + hs_ref[...]) + b_ref[...][None, :]
    cols = lax.broadcasted_iota(jnp.int32, (_BT, D_OUT_PAD), 1)
    um = jnp.where(cols < D_OUT, u, -1e30)
    m = jnp.max(um, axis=-1, keepdims=True)
    e = jnp.where(cols < D_OUT, jnp.exp(um - m), 0.0)
    lse = m + jnp.log(jnp.sum(e, axis=-1, keepdims=True))
    out_ref[...] = um - lse


def _tc_final(s_part, hs3, dinv, b3p):
    grid = (N_PAD // _BT,)
    return pl.pallas_call(
        _tc_final_body,
        grid=grid,
        in_specs=[
            pl.BlockSpec((NC, _BT, D_OUT_PAD), lambda i: (0, i, 0)),
            pl.BlockSpec((_BT, D_OUT_PAD), lambda i: (i, 0)),
            pl.BlockSpec((_BT,), lambda i: (i,)),
            pl.BlockSpec((D_OUT_PAD,), lambda i: (0,)),
        ],
        out_specs=pl.BlockSpec((_BT, D_OUT_PAD), lambda i: (i, 0)),
        out_shape=jax.ShapeDtypeStruct((N_PAD, D_OUT_PAD), jnp.float32),
    )(s_part, hs3, dinv, b3p)


def kernel(x, edge_index, W1, b1, W2, b2, W3, b3):
    e = edge_index.shape[1]
    t_chunks = -(-e // (NW * CH))
    t_chunks += t_chunks % 2
    per_w = t_chunks * CH
    e_pad = NW * per_w
    src = edge_index[0].astype(jnp.int32)
    dst = edge_index[1].astype(jnp.int32)
    src = jnp.concatenate([src, jnp.full((e_pad - e,), N, jnp.int32)])
    dst = jnp.concatenate([dst, jnp.full((e_pad - e,), N, jnp.int32)])
    pk_r = ((src << 14) | dst).reshape(NW, t_chunks, CH)

    z128 = jnp.zeros((N_PAD, D_HID), jnp.float32)
    ones = jnp.ones((CH, 128), jnp.float32)

    deg_part = _make_deg_kernel(t_chunks)(pk_r, ones, z128)
    dp0 = deg_part[0, :, 0]
    dp1 = deg_part[1, :, 0]

    x_pad = jnp.pad(x, ((0, N_PAD - N), (0, 0)))
    hs1, dinv = _tc1(x_pad, W1, dp0, dp1)

    seg128 = _make_seg_kernel(t_chunks, D_HID)
    s1 = seg128(hs1, pk_r, z128)
    hs2 = _tc_layer(s1, hs1, dinv, b1, W2)
    s2 = seg128(hs2, pk_r, z128)
    w3p = jnp.pad(W3, ((0, 0), (0, D_OUT_PAD - D_OUT)))
    b3p = jnp.pad(b3, (0, D_OUT_PAD - D_OUT))
    hs3 = _tc_layer(s2, hs2, dinv, b2, w3p)
    s3 = _make_seg_kernel(t_chunks, D_OUT_PAD)(hs3, pk_r, z128)
    logits = _tc_final(s3, hs3, dinv, b3p)
    return logits[:N, :D_OUT]


# speedup vs baseline: 2.1154x; 2.1154x over previous
import functools

import jax
import jax.numpy as jnp
from jax import lax
from jax.experimental import pallas as pl
from jax.experimental.pallas import tpu as pltpu, tpu_sc as plsc

N = 10000
D_IN = 128
D_HID = 128
D_OUT = 40
D_OUT_PAD = 128

NC = 2
NS = 16
NW = NC * NS
CH = 128

N_PAD = 10240
ROWS_PER_TILE = N_PAD // NS

_MESH = plsc.VectorSubcoreMesh(
    core_axis_name="c", subcore_axis_name="s", num_cores=NC, num_subcores=NS
)


def _seg_body(t0, t1, d, hs_hbm, src_hbm, dst_hbm, z_hbm, out_hbm,
              src_v, dst_v, buf, sem, acc):
    c = lax.axis_index("c")
    s = lax.axis_index("s")
    w = c * NS + s
    r0 = s * ROWS_PER_TILE
    tc = jnp.where(c == 0, t0, t1)
    pltpu.sync_copy(z_hbm.at[pl.ds(r0, ROWS_PER_TILE)],
                    acc.at[pl.ds(r0, ROWS_PER_TILE)])
    pltpu.sync_copy(src_hbm.at[w], src_v)
    pltpu.sync_copy(dst_hbm.at[w], dst_v)
    plsc.subcore_barrier()

    def body(j, carry):
        pltpu.async_copy(hs_hbm.at[src_v.at[j]], buf, sem).wait()
        pltpu.sync_copy(buf, acc.at[dst_v.at[j]], add=True)
        return carry

    lax.fori_loop(0, tc, body, 0, unroll=False)
    plsc.subcore_barrier()
    pltpu.sync_copy(acc.at[pl.ds(r0, ROWS_PER_TILE)],
                    out_hbm.at[c, pl.ds(r0, ROWS_PER_TILE)])


def _make_seg_kernel(t0, t1, d):
    t_max = max(t0, t1)
    return pl.kernel(
        functools.partial(_seg_body, t0, t1, d),
        out_type=jax.ShapeDtypeStruct((NC, N_PAD, d), jnp.float32),
        mesh=_MESH,
        scratch_types=[
            pltpu.VMEM((t_max, CH), jnp.int32),
            pltpu.VMEM((t_max, CH), jnp.int32),
            pltpu.VMEM((CH, d), jnp.float32),
            pltpu.SemaphoreType.DMA,
            pltpu.VMEM_SHARED((N_PAD, d), jnp.float32),
        ],
    )


def _deg_body(t0, t1, dst_hbm, ones_hbm, z_hbm, out_hbm,
              dst_v, ones_v, acc):
    c = lax.axis_index("c")
    s = lax.axis_index("s")
    w = c * NS + s
    r0 = s * ROWS_PER_TILE
    tc = jnp.where(c == 0, t0, t1)
    pltpu.sync_copy(z_hbm.at[pl.ds(r0, ROWS_PER_TILE)],
                    acc.at[pl.ds(r0, ROWS_PER_TILE)])
    pltpu.sync_copy(ones_hbm, ones_v)
    pltpu.sync_copy(dst_hbm.at[w], dst_v)
    plsc.subcore_barrier()

    def body(j, carry):
        pltpu.sync_copy(ones_v, acc.at[dst_v.at[j]], add=True)
        return carry

    lax.fori_loop(0, tc, body, 0, unroll=False)
    plsc.subcore_barrier()
    pltpu.sync_copy(acc.at[pl.ds(r0, ROWS_PER_TILE)],
                    out_hbm.at[c, pl.ds(r0, ROWS_PER_TILE)])


def _make_deg_kernel(t0, t1):
    t_max = max(t0, t1)
    return pl.kernel(
        functools.partial(_deg_body, t0, t1),
        out_type=jax.ShapeDtypeStruct((NC, N_PAD, 128), jnp.float32),
        mesh=_MESH,
        scratch_types=[
            pltpu.VMEM((t_max, CH), jnp.int32),
            pltpu.VMEM((CH, 128), jnp.float32),
            pltpu.VMEM_SHARED((N_PAD, 128), jnp.float32),
        ],
    )


_BT = 1024


def _tc1_body(x_ref, w_ref, dp0_ref, dp1_ref, hs_ref, dinv_ref):
    i = pl.program_id(0)
    deg = dp0_ref[...] + dp1_ref[...] + 1.0
    dinv = lax.rsqrt(deg)
    rows = i * _BT + lax.broadcasted_iota(jnp.int32, (_BT,), 0)
    mask = (rows < N).astype(jnp.float32)
    h = jnp.dot(x_ref[...], w_ref[...], preferred_element_type=jnp.float32)
    hs_ref[...] = h * (dinv * mask)[:, None]
    dinv_ref[...] = dinv


def _tc1(x_pad, w1, dp0, dp1):
    grid = (N_PAD // _BT,)
    return pl.pallas_call(
        _tc1_body,
        grid=grid,
        in_specs=[
            pl.BlockSpec((_BT, D_IN), lambda i: (i, 0)),
            pl.BlockSpec((D_IN, D_HID), lambda i: (0, 0)),
            pl.BlockSpec((_BT,), lambda i: (i,)),
            pl.BlockSpec((_BT,), lambda i: (i,)),
        ],
        out_specs=[
            pl.BlockSpec((_BT, D_HID), lambda i: (i, 0)),
            pl.BlockSpec((_BT,), lambda i: (i,)),
        ],
        out_shape=[
            jax.ShapeDtypeStruct((N_PAD, D_HID), jnp.float32),
            jax.ShapeDtypeStruct((N_PAD,), jnp.float32),
        ],
    )(x_pad, w1, dp0, dp1)


def _tc_layer_body(s_ref, hs_ref, dinv_ref, b_ref, w_ref, out_ref):
    i = pl.program_id(0)
    dinv = dinv_ref[...]
    u = dinv[:, None] * (s_ref[0] + s_ref[1] + hs_ref[...]) + b_ref[...][None, :]
    x2 = jnp.maximum(u, 0.0)
    rows = i * _BT + lax.broadcasted_iota(jnp.int32, (_BT,), 0)
    mask = (rows < N).astype(jnp.float32)
    h = jnp.dot(x2, w_ref[...], preferred_element_type=jnp.float32)
    out_ref[...] = h * (dinv * mask)[:, None]


def _tc_layer(s_part, hs_prev, dinv, b, w):
    d_in = hs_prev.shape[1]
    d_out = w.shape[1]
    grid = (N_PAD // _BT,)
    return pl.pallas_call(
        _tc_layer_body,
        grid=grid,
        in_specs=[
            pl.BlockSpec((NC, _BT, d_in), lambda i: (0, i, 0)),
            pl.BlockSpec((_BT, d_in), lambda i: (i, 0)),
            pl.BlockSpec((_BT,), lambda i: (i,)),
            pl.BlockSpec((d_in,), lambda i: (0,)),
            pl.BlockSpec((d_in, d_out), lambda i: (0, 0)),
        ],
        out_specs=pl.BlockSpec((_BT, d_out), lambda i: (i, 0)),
        out_shape=jax.ShapeDtypeStruct((N_PAD, d_out), jnp.float32),
    )(s_part, hs_prev, dinv, b, w)


def _tc_final_body(s_ref, hs_ref, dinv_ref, b_ref, out_ref):
    dinv = dinv_ref[...]
    u = dinv[:, None] * (s_ref[0] + s_ref[1] + hs_ref[...]) + b_ref[...][None, :]
    cols = lax.broadcasted_iota(jnp.int32, (_BT, D_OUT_PAD), 1)
    um = jnp.where(cols < D_OUT, u, -1e30)
    m = jnp.max(um, axis=-1, keepdims=True)
    e = jnp.where(cols < D_OUT, jnp.exp(um - m), 0.0)
    lse = m + jnp.log(jnp.sum(e, axis=-1, keepdims=True))
    out_ref[...] = um - lse


def _tc_final(s_part, hs3, dinv, b3p):
    grid = (N_PAD // _BT,)
    return pl.pallas_call(
        _tc_final_body,
        grid=grid,
        in_specs=[
            pl.BlockSpec((NC, _BT, D_OUT_PAD), lambda i: (0, i, 0)),
            pl.BlockSpec((_BT, D_OUT_PAD), lambda i: (i, 0)),
            pl.BlockSpec((_BT,), lambda i: (i,)),
            pl.BlockSpec((D_OUT_PAD,), lambda i: (0,)),
        ],
        out_specs=pl.BlockSpec((_BT, D_OUT_PAD), lambda i: (i, 0)),
        out_shape=jax.ShapeDtypeStruct((N_PAD, D_OUT_PAD), jnp.float32),
    )(s_part, hs3, dinv, b3p)


_F0 = 0.6


def _split_edges(idx, t_tot, t0, t1):
    t_max = max(t0, t1)
    e0 = NS * t0 * CH
    part0 = idx[:e0].reshape(NS, t0, CH)
    part1 = idx[e0:].reshape(NS, t1, CH)
    part0 = jnp.pad(part0, ((0, 0), (0, t_max - t0), (0, 0)), constant_values=N)
    part1 = jnp.pad(part1, ((0, 0), (0, t_max - t1), (0, 0)), constant_values=N)
    return jnp.concatenate([part0, part1], axis=0)


def kernel(x, edge_index, W1, b1, W2, b2, W3, b3):
    e = edge_index.shape[1]
    t_tot = -(-e // (NS * CH))
    t0 = max(1, min(t_tot - 1, round(t_tot * _F0)))
    t1 = t_tot - t0
    e_pad = NS * t_tot * CH
    src = edge_index[0].astype(jnp.int32)
    dst = edge_index[1].astype(jnp.int32)
    src = jnp.concatenate([src, jnp.full((e_pad - e,), N, jnp.int32)])
    dst = jnp.concatenate([dst, jnp.full((e_pad - e,), N, jnp.int32)])
    src_r = _split_edges(src, t_tot, t0, t1)
    dst_r = _split_edges(dst, t_tot, t0, t1)

    z128 = jnp.zeros((N_PAD, D_HID), jnp.float32)
    ones = jnp.ones((CH, 128), jnp.float32)

    deg_part = _make_deg_kernel(t0, t1)(dst_r, ones, z128)
    dp0 = deg_part[0, :, 0]
    dp1 = deg_part[1, :, 0]

    x_pad = jnp.pad(x, ((0, N_PAD - N), (0, 0)))
    hs1, dinv = _tc1(x_pad, W1, dp0, dp1)

    seg128 = _make_seg_kernel(t0, t1, D_HID)
    s1 = seg128(hs1, src_r, dst_r, z128)
    hs2 = _tc_layer(s1, hs1, dinv, b1, W2)
    s2 = seg128(hs2, src_r, dst_r, z128)
    w3p = jnp.pad(W3, ((0, 0), (0, D_OUT_PAD - D_OUT)))
    b3p = jnp.pad(b3, (0, D_OUT_PAD - D_OUT))
    hs3 = _tc_layer(s2, hs2, dinv, b2, w3p)
    s3 = _make_seg_kernel(t0, t1, D_OUT_PAD)(hs3, src_r, dst_r, z128)
    logits = _tc_final(s3, hs3, dinv, b3p)
    return logits[:N, :D_OUT]


# speedup vs baseline: 2.1512x; 1.0169x over previous
import functools

import jax
import jax.numpy as jnp
from jax import lax
from jax.experimental import pallas as pl
from jax.experimental.pallas import tpu as pltpu, tpu_sc as plsc

N = 10000
D_IN = 128
D_HID = 128
D_OUT = 40
D_OUT_PAD = 128

NC = 2
NS = 16
NW = NC * NS
CH = 128

N_PAD = 10240
ROWS_PER_TILE = N_PAD // NS

_MESH = plsc.VectorSubcoreMesh(
    core_axis_name="c", subcore_axis_name="s", num_cores=NC, num_subcores=NS
)


def _seg_body(t0, t1, d, hs_hbm, src_hbm, dst_hbm, z_hbm, out_hbm,
              src_v, dst_v, buf, sem, acc):
    c = lax.axis_index("c")
    s = lax.axis_index("s")
    w = c * NS + s
    r0 = s * ROWS_PER_TILE
    tc = jnp.where(c == 0, t0, t1)
    pltpu.sync_copy(z_hbm.at[pl.ds(r0, ROWS_PER_TILE)],
                    acc.at[pl.ds(r0, ROWS_PER_TILE)])
    pltpu.sync_copy(src_hbm.at[w], src_v)
    pltpu.sync_copy(dst_hbm.at[w], dst_v)
    plsc.subcore_barrier()

    def body(j, carry):
        pltpu.async_copy(hs_hbm.at[src_v.at[j]], buf, sem).wait()
        pltpu.sync_copy(buf, acc.at[dst_v.at[j]], add=True)
        return carry

    lax.fori_loop(0, tc, body, 0, unroll=False)
    plsc.subcore_barrier()
    pltpu.sync_copy(acc.at[pl.ds(r0, ROWS_PER_TILE)],
                    out_hbm.at[c, pl.ds(r0, ROWS_PER_TILE)])


def _make_seg_kernel(t0, t1, d):
    t_max = max(t0, t1)
    return pl.kernel(
        functools.partial(_seg_body, t0, t1, d),
        out_type=jax.ShapeDtypeStruct((NC, N_PAD, d), jnp.float32),
        mesh=_MESH,
        scratch_types=[
            pltpu.VMEM((t_max, CH), jnp.int32),
            pltpu.VMEM((t_max, CH), jnp.int32),
            pltpu.VMEM((CH, d), jnp.float32),
            pltpu.SemaphoreType.DMA,
            pltpu.VMEM_SHARED((N_PAD, d), jnp.float32),
        ],
    )


def _deg_body(t0, t1, dst_hbm, ones_hbm, z_hbm, out_hbm,
              dst_v, ones_v, acc):
    c = lax.axis_index("c")
    s = lax.axis_index("s")
    w = c * NS + s
    r0 = s * ROWS_PER_TILE
    tc = jnp.where(c == 0, t0, t1)
    pltpu.sync_copy(z_hbm.at[pl.ds(r0, ROWS_PER_TILE)],
                    acc.at[pl.ds(r0, ROWS_PER_TILE)])
    pltpu.sync_copy(ones_hbm, ones_v)
    pltpu.sync_copy(dst_hbm.at[w], dst_v)
    plsc.subcore_barrier()

    def body(j, carry):
        pltpu.sync_copy(ones_v, acc.at[dst_v.at[j]], add=True)
        return carry

    lax.fori_loop(0, tc, body, 0, unroll=False)
    plsc.subcore_barrier()
    pltpu.sync_copy(acc.at[pl.ds(r0, ROWS_PER_TILE)],
                    out_hbm.at[c, pl.ds(r0, ROWS_PER_TILE)])


def _make_deg_kernel(t0, t1):
    t_max = max(t0, t1)
    return pl.kernel(
        functools.partial(_deg_body, t0, t1),
        out_type=jax.ShapeDtypeStruct((NC, N_PAD, 128), jnp.float32),
        mesh=_MESH,
        scratch_types=[
            pltpu.VMEM((t_max, CH), jnp.int32),
            pltpu.VMEM((CH, 128), jnp.float32),
            pltpu.VMEM_SHARED((N_PAD, 128), jnp.float32),
        ],
    )


_BT = 1024


def _tc1_body(x_ref, w_ref, dp0_ref, dp1_ref, hs_ref, dinv_ref):
    i = pl.program_id(0)
    deg = dp0_ref[...] + dp1_ref[...] + 1.0
    dinv = lax.rsqrt(deg)
    rows = i * _BT + lax.broadcasted_iota(jnp.int32, (_BT,), 0)
    mask = (rows < N).astype(jnp.float32)
    h = jnp.dot(x_ref[...], w_ref[...], preferred_element_type=jnp.float32)
    hs_ref[...] = h * (dinv * mask)[:, None]
    dinv_ref[...] = dinv


def _tc1(x_pad, w1, dp0, dp1):
    grid = (N_PAD // _BT,)
    return pl.pallas_call(
        _tc1_body,
        grid=grid,
        in_specs=[
            pl.BlockSpec((_BT, D_IN), lambda i: (i, 0)),
            pl.BlockSpec((D_IN, D_HID), lambda i: (0, 0)),
            pl.BlockSpec((_BT,), lambda i: (i,)),
            pl.BlockSpec((_BT,), lambda i: (i,)),
        ],
        out_specs=[
            pl.BlockSpec((_BT, D_HID), lambda i: (i, 0)),
            pl.BlockSpec((_BT,), lambda i: (i,)),
        ],
        out_shape=[
            jax.ShapeDtypeStruct((N_PAD, D_HID), jnp.float32),
            jax.ShapeDtypeStruct((N_PAD,), jnp.float32),
        ],
    )(x_pad, w1, dp0, dp1)


def _tc_layer_body(s_ref, hs_ref, dinv_ref, b_ref, w_ref, out_ref):
    i = pl.program_id(0)
    dinv = dinv_ref[...]
    u = dinv[:, None] * (s_ref[0] + s_ref[1] + hs_ref[...]) + b_ref[...][None, :]
    x2 = jnp.maximum(u, 0.0)
    rows = i * _BT + lax.broadcasted_iota(jnp.int32, (_BT,), 0)
    mask = (rows < N).astype(jnp.float32)
    h = jnp.dot(x2, w_ref[...], preferred_element_type=jnp.float32)
    out_ref[...] = h * (dinv * mask)[:, None]


def _tc_layer(s_part, hs_prev, dinv, b, w):
    d_in = hs_prev.shape[1]
    d_out = w.shape[1]
    grid = (N_PAD // _BT,)
    return pl.pallas_call(
        _tc_layer_body,
        grid=grid,
        in_specs=[
            pl.BlockSpec((NC, _BT, d_in), lambda i: (0, i, 0)),
            pl.BlockSpec((_BT, d_in), lambda i: (i, 0)),
            pl.BlockSpec((_BT,), lambda i: (i,)),
            pl.BlockSpec((d_in,), lambda i: (0,)),
            pl.BlockSpec((d_in, d_out), lambda i: (0, 0)),
        ],
        out_specs=pl.BlockSpec((_BT, d_out), lambda i: (i, 0)),
        out_shape=jax.ShapeDtypeStruct((N_PAD, d_out), jnp.float32),
    )(s_part, hs_prev, dinv, b, w)


def _tc_final_body(s_ref, hs_ref, dinv_ref, b_ref, out_ref):
    dinv = dinv_ref[...]
    u = dinv[:, None] * (s_ref[0] + s_ref[1] + hs_ref[...]) + b_ref[...][None, :]
    cols = lax.broadcasted_iota(jnp.int32, (_BT, D_OUT_PAD), 1)
    um = jnp.where(cols < D_OUT, u, -1e30)
    m = jnp.max(um, axis=-1, keepdims=True)
    e = jnp.where(cols < D_OUT, jnp.exp(um - m), 0.0)
    lse = m + jnp.log(jnp.sum(e, axis=-1, keepdims=True))
    out_ref[...] = um - lse


def _tc_final(s_part, hs3, dinv, b3p):
    grid = (N_PAD // _BT,)
    return pl.pallas_call(
        _tc_final_body,
        grid=grid,
        in_specs=[
            pl.BlockSpec((NC, _BT, D_OUT_PAD), lambda i: (0, i, 0)),
            pl.BlockSpec((_BT, D_OUT_PAD), lambda i: (i, 0)),
            pl.BlockSpec((_BT,), lambda i: (i,)),
            pl.BlockSpec((D_OUT_PAD,), lambda i: (0,)),
        ],
        out_specs=pl.BlockSpec((_BT, D_OUT_PAD), lambda i: (i, 0)),
        out_shape=jax.ShapeDtypeStruct((N_PAD, D_OUT_PAD), jnp.float32),
    )(s_part, hs3, dinv, b3p)


_F0 = 0.585


def _split_edges(idx, t_tot, t0, t1):
    t_max = max(t0, t1)
    e0 = NS * t0 * CH
    part0 = idx[:e0].reshape(NS, t0, CH)
    part1 = idx[e0:].reshape(NS, t1, CH)
    part0 = jnp.pad(part0, ((0, 0), (0, t_max - t0), (0, 0)), constant_values=N)
    part1 = jnp.pad(part1, ((0, 0), (0, t_max - t1), (0, 0)), constant_values=N)
    return jnp.concatenate([part0, part1], axis=0)


def kernel(x, edge_index, W1, b1, W2, b2, W3, b3):
    e = edge_index.shape[1]
    t_tot = -(-e // (NS * CH))
    t0 = max(1, min(t_tot - 1, round(t_tot * _F0)))
    t1 = t_tot - t0
    e_pad = NS * t_tot * CH
    src = edge_index[0].astype(jnp.int32)
    dst = edge_index[1].astype(jnp.int32)
    src = jnp.concatenate([src, jnp.full((e_pad - e,), N, jnp.int32)])
    dst = jnp.concatenate([dst, jnp.full((e_pad - e,), N, jnp.int32)])
    src_r = _split_edges(src, t_tot, t0, t1)
    dst_r = _split_edges(dst, t_tot, t0, t1)
    t0d = t_tot // 2
    t1d = t_tot - t0d
    dst_deg = _split_edges(dst, t_tot, t0d, t1d)

    z128 = jnp.zeros((N_PAD, D_HID), jnp.float32)
    ones = jnp.ones((CH, 128), jnp.float32)

    deg_part = _make_deg_kernel(t0d, t1d)(dst_deg, ones, z128)
    dp0 = deg_part[0, :, 0]
    dp1 = deg_part[1, :, 0]

    x_pad = jnp.pad(x, ((0, N_PAD - N), (0, 0)))
    hs1, dinv = _tc1(x_pad, W1, dp0, dp1)

    seg128 = _make_seg_kernel(t0, t1, D_HID)
    s1 = seg128(hs1, src_r, dst_r, z128)
    hs2 = _tc_layer(s1, hs1, dinv, b1, W2)
    s2 = seg128(hs2, src_r, dst_r, z128)
    w3p = jnp.pad(W3, ((0, 0), (0, D_OUT_PAD - D_OUT)))
    b3p = jnp.pad(b3, (0, D_OUT_PAD - D_OUT))
    hs3 = _tc_layer(s2, hs2, dinv, b2, w3p)
    s3 = _make_seg_kernel(t0, t1, D_OUT_PAD)(hs3, src_r, dst_r, z128)
    logits = _tc_final(s3, hs3, dinv, b3p)
    return logits[:N, :D_OUT]
